# baseline (speedup 1.0000x reference)
import jax
import jax.numpy as jnp
from jax import lax
from jax.experimental import pallas as pl
from jax.experimental.pallas import tpu as pltpu
from jax.experimental.pallas import tpu_sc as plsc

S = 2048
H = 16
MAX_REL = 32
NUM_TABLE_ROWS = 2 * MAX_REL + 1

NC = 2
NS = 16
NW = NC * NS
RPW = S // NW

CHUNK = 256
NCHUNK = S // CHUNK
CQ = 287
BP_ROWS = 576
Q0_MAX = BP_ROWS - CHUNK

PIPE_ROWS = 1


def _sc_body(table_hbm, out_hbm, table_v, bp_v, sem):
    wid = lax.axis_index("s") * NC + lax.axis_index("c")
    base = wid * RPW

    pltpu.sync_copy(table_hbm, table_v)

    def build(q, carry):
        idx = jnp.clip(CQ - q, -MAX_REL, MAX_REL) + MAX_REL
        bp_v[q, :] = table_v[idx, :]
        return carry

    lax.fori_loop(0, BP_ROWS, build, 0, unroll=4)

    def emit(r, carry):
        i = base + r
        for c in range(NCHUNK):
            q0 = jnp.clip(c * CHUNK - i + CQ, 0, Q0_MAX - 1)
            pltpu.make_async_copy(
                bp_v.at[pl.ds(q0, CHUNK), :],
                out_hbm.at[i, pl.ds(c * CHUNK, CHUNK), :],
                sem,
            ).start()

        @pl.when(r >= PIPE_ROWS)
        def _drain_prev_row():
            for c in range(NCHUNK):
                pltpu.make_async_copy(
                    bp_v.at[pl.ds(0, CHUNK), :],
                    out_hbm.at[base, pl.ds(c * CHUNK, CHUNK), :],
                    sem,
                ).wait()

        return carry

    lax.fori_loop(0, RPW, emit, 0)

    def drain(r, carry):
        for c in range(NCHUNK):
            pltpu.make_async_copy(
                bp_v.at[pl.ds(0, CHUNK), :],
                out_hbm.at[base, pl.ds(c * CHUNK, CHUNK), :],
                sem,
            ).wait()
        return carry

    lax.fori_loop(0, PIPE_ROWS, drain, 0)


def kernel(seq_len, relative_attention_bias):
    mesh = plsc.VectorSubcoreMesh(core_axis_name="c", subcore_axis_name="s")
    out = pl.kernel(
        _sc_body,
        mesh=mesh,
        out_type=jax.ShapeDtypeStruct((S, S, H), jnp.float32),
        scratch_types=[
            pltpu.VMEM((NUM_TABLE_ROWS, H), jnp.float32),
            pltpu.VMEM((BP_ROWS, H), jnp.float32),
            pltpu.SemaphoreType.DMA,
        ],
        compiler_params=pltpu.CompilerParams(use_tc_tiling_on_sc=True),
    )(relative_attention_bias)
    return out


# speedup vs baseline: 1.0083x; 1.0083x over previous
import jax
import jax.numpy as jnp
from jax import lax
from jax.experimental import pallas as pl
from jax.experimental.pallas import tpu as pltpu
from jax.experimental.pallas import tpu_sc as plsc

S = 2048
H = 16
MAX_REL = 32
NUM_TABLE_ROWS = 2 * MAX_REL + 1

NC = 2
NS = 16
NW = NC * NS

G_ROWS = 2 * S
STRIP = G_ROWS // NW

PIPE = 16


def _sc_grev_body(table_hbm, grev_hbm, table_v, strip_v):
    wid = lax.axis_index("s") * NC + lax.axis_index("c")
    base = wid * STRIP

    pltpu.sync_copy(table_hbm, table_v)

    def build(q, carry):
        idx = jnp.clip(S - 1 - (base + q), -MAX_REL, MAX_REL) + MAX_REL
        strip_v[q, :] = table_v[idx, :]
        return carry

    lax.fori_loop(0, STRIP, build, 0, unroll=4)
    pltpu.sync_copy(strip_v, grev_hbm.at[pl.ds(base, STRIP), :])


def _tc_emit_body(grev_ref, out_ref, sem):
    def emit(i, carry):
        pltpu.make_async_copy(
            grev_ref.at[pl.ds(S - 1 - i, S), :], out_ref.at[i], sem
        ).start()

        @pl.when(i >= PIPE)
        def _drain_one():
            pltpu.make_async_copy(
                grev_ref.at[pl.ds(0, S), :], out_ref.at[0], sem
            ).wait()

        return carry

    lax.fori_loop(0, S, emit, 0)

    def drain(i, carry):
        pltpu.make_async_copy(
            grev_ref.at[pl.ds(0, S), :], out_ref.at[0], sem
        ).wait()
        return carry

    lax.fori_loop(0, PIPE, drain, 0)


def kernel(seq_len, relative_attention_bias):
    mesh = plsc.VectorSubcoreMesh(core_axis_name="c", subcore_axis_name="s")
    grev = pl.kernel(
        _sc_grev_body,
        mesh=mesh,
        out_type=jax.ShapeDtypeStruct((G_ROWS, H), jnp.float32),
        scratch_types=[
            pltpu.VMEM((NUM_TABLE_ROWS, H), jnp.float32),
            pltpu.VMEM((STRIP, H), jnp.float32),
        ],
        compiler_params=pltpu.CompilerParams(use_tc_tiling_on_sc=True),
    )(relative_attention_bias)

    out = pl.pallas_call(
        _tc_emit_body,
        out_shape=jax.ShapeDtypeStruct((S, S, H), jnp.float32),
        in_specs=[pl.BlockSpec(memory_space=pltpu.VMEM)],
        out_specs=pl.BlockSpec(memory_space=pl.ANY),
        scratch_shapes=[pltpu.SemaphoreType.DMA],
    )(grev)
    return out


# speedup vs baseline: 2.1551x; 2.1374x over previous
import jax
import jax.numpy as jnp
from jax import lax
from jax.experimental import pallas as pl
from jax.experimental.pallas import tpu as pltpu
from jax.experimental.pallas import tpu_sc as plsc

S = 2048
H = 16
MAX_REL = 32
NUM_TABLE_ROWS = 2 * MAX_REL + 1

NC = 2
NS = 16
NW = NC * NS

G_COLS = 2 * S
STRIP = G_COLS // NW
CHUNKS = STRIP // 16

PIPE = 16


def _sc_grevt_body(table_t_hbm, grevt_hbm, table_t_v, strip_v):
    wid = lax.axis_index("s") * NC + lax.axis_index("c")
    t_base = wid * STRIP

    pltpu.sync_copy(table_t_hbm, table_t_v)

    def build(n, carry):
        h = n // CHUNKS
        t0 = t_base + (n % CHUNKS) * 16
        idx = jnp.clip((S - 1 + MAX_REL) - t0 - lax.iota(jnp.int32, 16), 0, 2 * MAX_REL)
        vals = plsc.load_gather(table_t_v, [jnp.zeros((16,), jnp.int32) + h, idx])
        strip_v[h, pl.ds((n % CHUNKS) * 16, 16)] = vals
        return carry

    lax.fori_loop(0, H * CHUNKS, build, 0)

    pltpu.sync_copy(
        strip_v.at[pl.ds(0, 8), :], grevt_hbm.at[pl.ds(0, 8), pl.ds(t_base, STRIP)]
    )
    pltpu.sync_copy(
        strip_v.at[pl.ds(8, 8), :], grevt_hbm.at[pl.ds(8, 8), pl.ds(t_base, STRIP)]
    )


def _tc_emit_body(grevt_ref, out_ref, stage_ref, sems):
    def emit(i, carry):
        slot = lax.rem(i, 2)

        @pl.when(i >= 2)
        def _reclaim_slot():
            pltpu.make_async_copy(
                stage_ref.at[slot], out_ref.at[0], sems.at[slot]
            ).wait()

        start = S - 1 - i
        res = lax.rem(start, 128)
        base = pl.multiple_of(start - res, 128)
        wide = grevt_ref[:, pl.ds(base, S + 128)]
        rolled = pltpu.roll(wide, lax.rem((S + 128) - res, S + 128), 1)
        stage_ref[slot] = rolled[:, :S]
        pltpu.make_async_copy(
            stage_ref.at[slot], out_ref.at[i], sems.at[slot]
        ).start()
        return carry

    lax.fori_loop(0, S, emit, 0)

    def drain(s, carry):
        pltpu.make_async_copy(
            stage_ref.at[s], out_ref.at[0], sems.at[s]
        ).wait()
        return carry

    lax.fori_loop(0, 2, drain, 0)


def kernel(seq_len, relative_attention_bias):
    table_t = relative_attention_bias.T

    mesh = plsc.VectorSubcoreMesh(core_axis_name="c", subcore_axis_name="s")
    grevt = pl.kernel(
        _sc_grevt_body,
        mesh=mesh,
        out_type=jax.ShapeDtypeStruct((H, G_COLS), jnp.float32),
        scratch_types=[
            pltpu.VMEM((H, NUM_TABLE_ROWS), jnp.float32),
            pltpu.VMEM((H, STRIP), jnp.float32),
        ],
        compiler_params=pltpu.CompilerParams(
            use_tc_tiling_on_sc=True, needs_layout_passes=False
        ),
    )(table_t)

    out_ihj = pl.pallas_call(
        _tc_emit_body,
        out_shape=jax.ShapeDtypeStruct((S, H, S), jnp.float32),
        in_specs=[pl.BlockSpec(memory_space=pltpu.VMEM)],
        out_specs=pl.BlockSpec(memory_space=pl.ANY),
        scratch_shapes=[
            pltpu.VMEM((2, H, S), jnp.float32),
            pltpu.SemaphoreType.DMA((2,)),
        ],
    )(grevt)
    return out_ihj.transpose(0, 2, 1)


# speedup vs baseline: 13.4837x; 6.2566x over previous
import jax
import jax.numpy as jnp
from jax import lax
from jax.experimental import pallas as pl
from jax.experimental.pallas import tpu as pltpu
from jax.experimental.pallas import tpu_sc as plsc

S = 2048
H = 16
MAX_REL = 32
NUM_TABLE_ROWS = 2 * MAX_REL + 1

NC = 2
NS = 16
NW = NC * NS

G_COLS = 2 * S
STRIP = G_COLS // NW
CHUNKS = STRIP // 16

PIPE = 16


def _sc_grevt_body(table_t_hbm, grevt_hbm, table_t_v, strip_v):
    wid = lax.axis_index("s") * NC + lax.axis_index("c")
    t_base = wid * STRIP

    pltpu.sync_copy(table_t_hbm, table_t_v)

    def build(n, carry):
        h = n // CHUNKS
        t0 = t_base + (n % CHUNKS) * 16
        idx = jnp.clip((S - 1 + MAX_REL) - t0 - lax.iota(jnp.int32, 16), 0, 2 * MAX_REL)
        vals = plsc.load_gather(table_t_v, [jnp.zeros((16,), jnp.int32) + h, idx])
        strip_v[h, pl.ds((n % CHUNKS) * 16, 16)] = vals
        return carry

    lax.fori_loop(0, H * CHUNKS, build, 0)

    pltpu.sync_copy(
        strip_v.at[pl.ds(0, 8), :], grevt_hbm.at[pl.ds(0, 8), pl.ds(t_base, STRIP)]
    )
    pltpu.sync_copy(
        strip_v.at[pl.ds(8, 8), :], grevt_hbm.at[pl.ds(8, 8), pl.ds(t_base, STRIP)]
    )


GS_COLS = 3968


def _tc_emit_body(grevt_ref, out_ref, gshift_ref, sem):
    def build(p, carry):
        rolled = pltpu.roll(grevt_ref[:, :], lax.rem(G_COLS - p, G_COLS), 1)
        gshift_ref[p] = rolled[:, :GS_COLS]
        return carry

    lax.fori_loop(0, 128, build, 0)

    def emit(i, carry):
        start = S - 1 - i
        res = lax.rem(start, 128)
        base = pl.multiple_of(start - res, 128)
        pltpu.make_async_copy(
            gshift_ref.at[res, :, pl.ds(base, S)], out_ref.at[i], sem
        ).start()

        @pl.when(i >= PIPE)
        def _drain_one():
            pltpu.make_async_copy(
                gshift_ref.at[0, :, pl.ds(0, S)], out_ref.at[0], sem
            ).wait()

        return carry

    lax.fori_loop(0, S, emit, 0)

    def drain(s, carry):
        pltpu.make_async_copy(
            gshift_ref.at[0, :, pl.ds(0, S)], out_ref.at[0], sem
        ).wait()
        return carry

    lax.fori_loop(0, PIPE, drain, 0)


def kernel(seq_len, relative_attention_bias):
    table_t = relative_attention_bias.T

    mesh = plsc.VectorSubcoreMesh(core_axis_name="c", subcore_axis_name="s")
    grevt = pl.kernel(
        _sc_grevt_body,
        mesh=mesh,
        out_type=jax.ShapeDtypeStruct((H, G_COLS), jnp.float32),
        scratch_types=[
            pltpu.VMEM((H, NUM_TABLE_ROWS), jnp.float32),
            pltpu.VMEM((H, STRIP), jnp.float32),
        ],
        compiler_params=pltpu.CompilerParams(
            use_tc_tiling_on_sc=True, needs_layout_passes=False
        ),
    )(table_t)

    out_ihj = pl.pallas_call(
        _tc_emit_body,
        out_shape=jax.ShapeDtypeStruct((S, H, S), jnp.float32),
        in_specs=[pl.BlockSpec(memory_space=pltpu.VMEM)],
        out_specs=pl.BlockSpec(memory_space=pl.ANY),
        scratch_shapes=[
            pltpu.VMEM((128, H, GS_COLS), jnp.float32),
            pltpu.SemaphoreType.DMA,
        ],
        compiler_params=pltpu.CompilerParams(
            vmem_limit_bytes=64 * 1024 * 1024
        ),
    )(grevt)
    return out_ihj.transpose(0, 2, 1)


# speedup vs baseline: 14.6811x; 1.0888x over previous
import jax
import jax.numpy as jnp
from jax import lax
from jax.experimental import pallas as pl
from jax.experimental.pallas import tpu as pltpu
from jax.experimental.pallas import tpu_sc as plsc

S = 2048
H = 16
MAX_REL = 32
NUM_TABLE_ROWS = 2 * MAX_REL + 1

NC = 2
NS = 16
NW = NC * NS

G_COLS = 2 * S
STRIP = G_COLS // NW
CHUNKS = STRIP // 16

PIPE = 16


def _sc_grevt_body(table_t_hbm, grevt_hbm, table_t_v, strip_v):
    wid = lax.axis_index("s") * NC + lax.axis_index("c")
    t_base = wid * STRIP

    pltpu.sync_copy(table_t_hbm, table_t_v)

    def build(n, carry):
        h = n // CHUNKS
        t0 = t_base + (n % CHUNKS) * 16
        idx = jnp.clip((S - 1 + MAX_REL) - t0 - lax.iota(jnp.int32, 16), 0, 2 * MAX_REL)
        vals = plsc.load_gather(table_t_v, [jnp.zeros((16,), jnp.int32) + h, idx])
        strip_v[h, pl.ds((n % CHUNKS) * 16, 16)] = vals
        return carry

    lax.fori_loop(0, H * CHUNKS, build, 0)

    pltpu.sync_copy(
        strip_v.at[pl.ds(0, 8), :], grevt_hbm.at[pl.ds(0, 8), pl.ds(t_base, STRIP)]
    )
    pltpu.sync_copy(
        strip_v.at[pl.ds(8, 8), :], grevt_hbm.at[pl.ds(8, 8), pl.ds(t_base, STRIP)]
    )


GS_COLS = 3968


def _tc_emit_body(grevt_ref, out_ref, gshift_ref, sem):
    ROWS_PER_RES = S // 128

    def build_and_emit(p, carry):
        rolled = pltpu.roll(grevt_ref[:, :], lax.rem(G_COLS - p, G_COLS), 1)
        gshift_ref[p] = rolled[:, :GS_COLS]

        for k in range(ROWS_PER_RES):
            base = 128 * k
            i = S - 1 - p - base
            pltpu.make_async_copy(
                gshift_ref.at[p, :, pl.ds(base, S)], out_ref.at[i], sem
            ).start()

        @pl.when(p >= 1)
        def _drain_prev():
            for _ in range(ROWS_PER_RES):
                pltpu.make_async_copy(
                    gshift_ref.at[0, :, pl.ds(0, S)], out_ref.at[0], sem
                ).wait()

        return carry

    lax.fori_loop(0, 128, build_and_emit, 0)

    def drain(s, carry):
        for _ in range(ROWS_PER_RES):
            pltpu.make_async_copy(
                gshift_ref.at[0, :, pl.ds(0, S)], out_ref.at[0], sem
            ).wait()
        return carry

    lax.fori_loop(0, 1, drain, 0)


def kernel(seq_len, relative_attention_bias):
    table_t = relative_attention_bias.T

    mesh = plsc.VectorSubcoreMesh(core_axis_name="c", subcore_axis_name="s")
    grevt = pl.kernel(
        _sc_grevt_body,
        mesh=mesh,
        out_type=jax.ShapeDtypeStruct((H, G_COLS), jnp.float32),
        scratch_types=[
            pltpu.VMEM((H, NUM_TABLE_ROWS), jnp.float32),
            pltpu.VMEM((H, STRIP), jnp.float32),
        ],
        compiler_params=pltpu.CompilerParams(
            use_tc_tiling_on_sc=True, needs_layout_passes=False
        ),
    )(table_t)

    out_ihj = pl.pallas_call(
        _tc_emit_body,
        out_shape=jax.ShapeDtypeStruct((S, H, S), jnp.float32),
        in_specs=[pl.BlockSpec(memory_space=pltpu.VMEM)],
        out_specs=pl.BlockSpec(memory_space=pl.ANY),
        scratch_shapes=[
            pltpu.VMEM((128, H, GS_COLS), jnp.float32),
            pltpu.SemaphoreType.DMA,
        ],
        compiler_params=pltpu.CompilerParams(
            vmem_limit_bytes=64 * 1024 * 1024
        ),
    )(grevt)
    return out_ihj.transpose(0, 2, 1)


# speedup vs baseline: 17.3754x; 1.1835x over previous
import jax
import jax.numpy as jnp
from jax import lax
from jax.experimental import pallas as pl
from jax.experimental.pallas import tpu as pltpu
from jax.experimental.pallas import tpu_sc as plsc

S = 2048
H = 16
MAX_REL = 32
NUM_TABLE_ROWS = 2 * MAX_REL + 1

NC = 2
NS = 16
NW = NC * NS

G_COLS = 2 * S
STRIP = G_COLS // NW
CHUNKS = STRIP // 16

PIPE = 16


def _sc_grevt_body(table_t_hbm, grevt_hbm, table_t_v, strip_v):
    wid = lax.axis_index("s") * NC + lax.axis_index("c")
    t_base = wid * STRIP

    pltpu.sync_copy(table_t_hbm, table_t_v)

    def build(n, carry):
        h = n // CHUNKS
        t0 = t_base + (n % CHUNKS) * 16
        idx = jnp.clip((S - 1 + MAX_REL) - t0 - lax.iota(jnp.int32, 16), 0, 2 * MAX_REL)
        vals = plsc.load_gather(table_t_v, [jnp.zeros((16,), jnp.int32) + h, idx])
        strip_v[h, pl.ds((n % CHUNKS) * 16, 16)] = vals
        return carry

    lax.fori_loop(0, H * CHUNKS, build, 0)

    pltpu.sync_copy(
        strip_v.at[pl.ds(0, 8), :], grevt_hbm.at[pl.ds(0, 8), pl.ds(t_base, STRIP)]
    )
    pltpu.sync_copy(
        strip_v.at[pl.ds(8, 8), :], grevt_hbm.at[pl.ds(8, 8), pl.ds(t_base, STRIP)]
    )


GS_COLS = 3968


def _tc_emit_body(grevt_ref, out_ref, gshift_ref, sem):
    ROWS_PER_RES = S // 128

    def build_and_emit(p, carry):
        rolled = pltpu.roll(grevt_ref[:, :], lax.rem(G_COLS - p, G_COLS), 1)
        gshift_ref[p] = rolled[:, :GS_COLS]

        for k in range(ROWS_PER_RES):
            base = 128 * k
            i = S - 1 - p - base
            pltpu.make_async_copy(
                gshift_ref.at[p, :, pl.ds(base, S)], out_ref.at[i], sem.at[k % 4]
            ).start()

        @pl.when(p >= 2)
        def _drain_prev():
            for k in range(ROWS_PER_RES):
                pltpu.make_async_copy(
                    gshift_ref.at[0, :, pl.ds(0, S)], out_ref.at[0], sem.at[k % 4]
                ).wait()

        return carry

    lax.fori_loop(0, 128, build_and_emit, 0)

    def drain(s, carry):
        for k in range(2 * ROWS_PER_RES):
            pltpu.make_async_copy(
                gshift_ref.at[0, :, pl.ds(0, S)], out_ref.at[0], sem.at[k % 4]
            ).wait()
        return carry

    lax.fori_loop(0, 1, drain, 0)


def kernel(seq_len, relative_attention_bias):
    table_t = relative_attention_bias.T

    mesh = plsc.VectorSubcoreMesh(core_axis_name="c", subcore_axis_name="s")
    grevt = pl.kernel(
        _sc_grevt_body,
        mesh=mesh,
        out_type=jax.ShapeDtypeStruct((H, G_COLS), jnp.float32),
        scratch_types=[
            pltpu.VMEM((H, NUM_TABLE_ROWS), jnp.float32),
            pltpu.VMEM((H, STRIP), jnp.float32),
        ],
        compiler_params=pltpu.CompilerParams(
            use_tc_tiling_on_sc=True, needs_layout_passes=False
        ),
    )(table_t)

    out_ihj = pl.pallas_call(
        _tc_emit_body,
        out_shape=jax.ShapeDtypeStruct((S, H, S), jnp.float32),
        in_specs=[pl.BlockSpec(memory_space=pltpu.VMEM)],
        out_specs=pl.BlockSpec(memory_space=pl.ANY),
        scratch_shapes=[
            pltpu.VMEM((128, H, GS_COLS), jnp.float32),
            pltpu.SemaphoreType.DMA((4,)),
        ],
        compiler_params=pltpu.CompilerParams(
            vmem_limit_bytes=64 * 1024 * 1024
        ),
    )(grevt)
    return out_ihj.transpose(0, 2, 1)


# speedup vs baseline: 17.4013x; 1.0015x over previous
import jax
import jax.numpy as jnp
from jax import lax
from jax.experimental import pallas as pl
from jax.experimental.pallas import tpu as pltpu
from jax.experimental.pallas import tpu_sc as plsc

S = 2048
H = 16
MAX_REL = 32
NUM_TABLE_ROWS = 2 * MAX_REL + 1

NC = 2
NS = 16
NW = NC * NS

G_COLS = 2 * S
STRIP = G_COLS // NW
CHUNKS = STRIP // 16

PIPE = 16


def _sc_grevt_body(table_t_hbm, grevt_hbm, table_t_v, strip_v):
    wid = lax.axis_index("s") * NC + lax.axis_index("c")
    t_base = wid * STRIP

    pltpu.sync_copy(table_t_hbm, table_t_v)

    def build(n, carry):
        h = n // CHUNKS
        t0 = t_base + (n % CHUNKS) * 16
        idx = jnp.clip((S - 1 + MAX_REL) - t0 - lax.iota(jnp.int32, 16), 0, 2 * MAX_REL)
        vals = plsc.load_gather(table_t_v, [jnp.zeros((16,), jnp.int32) + h, idx])
        strip_v[h, pl.ds((n % CHUNKS) * 16, 16)] = vals
        return carry

    lax.fori_loop(0, H * CHUNKS, build, 0)

    pltpu.sync_copy(
        strip_v.at[pl.ds(0, 8), :], grevt_hbm.at[pl.ds(0, 8), pl.ds(t_base, STRIP)]
    )
    pltpu.sync_copy(
        strip_v.at[pl.ds(8, 8), :], grevt_hbm.at[pl.ds(8, 8), pl.ds(t_base, STRIP)]
    )


GS_COLS = 3968


def _tc_emit_body(grevt_ref, out_ref, gshift_ref, sem):
    ROWS_PER_RES = S // 128

    def build_and_emit(p, carry):
        rolled = pltpu.roll(grevt_ref[:, :], lax.rem(G_COLS - p, G_COLS), 1)
        gshift_ref[p] = rolled[:, :GS_COLS]

        for k in range(ROWS_PER_RES):
            base = 128 * k
            i = S - 1 - p - base
            pltpu.make_async_copy(
                gshift_ref.at[p, :, pl.ds(base, S)], out_ref.at[i], sem.at[k % 8]
            ).start()

        @pl.when(p >= 3)
        def _drain_prev():
            for k in range(ROWS_PER_RES):
                pltpu.make_async_copy(
                    gshift_ref.at[0, :, pl.ds(0, S)], out_ref.at[0], sem.at[k % 8]
                ).wait()

        return carry

    lax.fori_loop(0, 128, build_and_emit, 0)

    def drain(s, carry):
        for k in range(3 * ROWS_PER_RES):
            pltpu.make_async_copy(
                gshift_ref.at[0, :, pl.ds(0, S)], out_ref.at[0], sem.at[k % 8]
            ).wait()
        return carry

    lax.fori_loop(0, 1, drain, 0)


def kernel(seq_len, relative_attention_bias):
    table_t = relative_attention_bias.T

    mesh = plsc.VectorSubcoreMesh(core_axis_name="c", subcore_axis_name="s")
    grevt = pl.kernel(
        _sc_grevt_body,
        mesh=mesh,
        out_type=jax.ShapeDtypeStruct((H, G_COLS), jnp.float32),
        scratch_types=[
            pltpu.VMEM((H, NUM_TABLE_ROWS), jnp.float32),
            pltpu.VMEM((H, STRIP), jnp.float32),
        ],
        compiler_params=pltpu.CompilerParams(
            use_tc_tiling_on_sc=True, needs_layout_passes=False
        ),
    )(table_t)

    out_ihj = pl.pallas_call(
        _tc_emit_body,
        out_shape=jax.ShapeDtypeStruct((S, H, S), jnp.float32),
        in_specs=[pl.BlockSpec(memory_space=pltpu.VMEM)],
        out_specs=pl.BlockSpec(memory_space=pl.ANY),
        scratch_shapes=[
            pltpu.VMEM((128, H, GS_COLS), jnp.float32),
            pltpu.SemaphoreType.DMA((8,)),
        ],
        compiler_params=pltpu.CompilerParams(
            vmem_limit_bytes=64 * 1024 * 1024
        ),
    )(grevt)
    return out_ihj.transpose(0, 2, 1)


# speedup vs baseline: 17.4198x; 1.0011x over previous
import jax
import jax.numpy as jnp
from jax import lax
from jax.experimental import pallas as pl
from jax.experimental.pallas import tpu as pltpu
from jax.experimental.pallas import tpu_sc as plsc

S = 2048
H = 16
MAX_REL = 32
NUM_TABLE_ROWS = 2 * MAX_REL + 1

NC = 2
NS = 16
NW = NC * NS

G_COLS = 2 * S
STRIP = G_COLS // NW
CHUNKS = STRIP // 16


def _sc_grevt_body(table_t_hbm, grevt_hbm, table_t_v, strip_v):
    wid = lax.axis_index("s") * NC + lax.axis_index("c")
    t_base = wid * STRIP

    pltpu.sync_copy(table_t_hbm, table_t_v)

    def build(n, carry):
        h = n // CHUNKS
        t0 = t_base + (n % CHUNKS) * 16
        idx = jnp.clip((S - 1 + MAX_REL) - t0 - lax.iota(jnp.int32, 16), 0, 2 * MAX_REL)
        vals = plsc.load_gather(table_t_v, [jnp.zeros((16,), jnp.int32) + h, idx])
        strip_v[h, pl.ds((n % CHUNKS) * 16, 16)] = vals
        return carry

    lax.fori_loop(0, H * CHUNKS, build, 0)

    pltpu.sync_copy(
        strip_v.at[pl.ds(0, 8), :], grevt_hbm.at[pl.ds(0, 8), pl.ds(t_base, STRIP)]
    )
    pltpu.sync_copy(
        strip_v.at[pl.ds(8, 8), :], grevt_hbm.at[pl.ds(8, 8), pl.ds(t_base, STRIP)]
    )


GS_COLS = 3968


def _tc_emit_body(grevt_ref, out_ref, gshift_ref, sem):
    ROWS_PER_RES = S // 128

    def build_and_emit(p, carry):
        rolled = pltpu.roll(grevt_ref[:, :], lax.rem(G_COLS - p, G_COLS), 1)
        gshift_ref[p] = rolled[:, :GS_COLS]

        for k in range(ROWS_PER_RES):
            base = 128 * k
            i = S - 1 - p - base
            pltpu.make_async_copy(
                gshift_ref.at[p, :, pl.ds(base, S)], out_ref.at[i], sem.at[k % 8]
            ).start()

        @pl.when(p >= 3)
        def _drain_prev():
            for k in range(ROWS_PER_RES):
                pltpu.make_async_copy(
                    gshift_ref.at[0, :, pl.ds(0, S)], out_ref.at[0], sem.at[k % 8]
                ).wait()

        return carry

    lax.fori_loop(0, 128, build_and_emit, 0)

    def drain(s, carry):
        for k in range(3 * ROWS_PER_RES):
            pltpu.make_async_copy(
                gshift_ref.at[0, :, pl.ds(0, S)], out_ref.at[0], sem.at[k % 8]
            ).wait()
        return carry

    lax.fori_loop(0, 1, drain, 0)


def kernel(seq_len, relative_attention_bias):
    table_t = relative_attention_bias.T

    mesh = plsc.VectorSubcoreMesh(core_axis_name="c", subcore_axis_name="s")
    grevt = pl.kernel(
        _sc_grevt_body,
        mesh=mesh,
        out_type=jax.ShapeDtypeStruct((H, G_COLS), jnp.float32),
        scratch_types=[
            pltpu.VMEM((H, NUM_TABLE_ROWS), jnp.float32),
            pltpu.VMEM((H, STRIP), jnp.float32),
        ],
        compiler_params=pltpu.CompilerParams(
            use_tc_tiling_on_sc=True, needs_layout_passes=False
        ),
    )(table_t)

    out_ihj = pl.pallas_call(
        _tc_emit_body,
        out_shape=jax.ShapeDtypeStruct((S, H, S), jnp.float32),
        in_specs=[pl.BlockSpec(memory_space=pltpu.VMEM)],
        out_specs=pl.BlockSpec(memory_space=pl.ANY),
        scratch_shapes=[
            pltpu.VMEM((128, H, GS_COLS), jnp.float32),
            pltpu.SemaphoreType.DMA((8,)),
        ],
        compiler_params=pltpu.CompilerParams(
            vmem_limit_bytes=64 * 1024 * 1024
        ),
    )(grevt)
    return out_ihj.transpose(0, 2, 1)
